# baseline (speedup 1.0000x reference)
import math

import jax
import jax.numpy as jnp
import numpy as np
from jax.experimental import pallas as pl
from jax.experimental.pallas import tpu as pltpu

_D = 768
_B = 4
_N = 576
_SIDE = 24
_KA = 144
_KC = 144
_NCTX = _N - _KA
_NITER = 10
_SW = 0.1

_HI = jax.lax.Precision.HIGHEST
_DEF = jax.lax.Precision.DEFAULT


def _dot(a, b, ca, cb, prec=_HI):
    return jax.lax.dot_general(
        a, b, (((ca,), (cb,)), ((), ())),
        precision=prec, preferred_element_type=jnp.float32)


def _iota_row(n, dtype=jnp.float32):
    return jax.lax.broadcasted_iota(jnp.int32, (1, n), 1).astype(dtype)


def _iota_col(n, dtype=jnp.float32):
    return jax.lax.broadcasted_iota(jnp.int32, (n, 1), 0).astype(dtype)


def _first_argmin_onehot(d, k):
    rmin = jnp.min(d, axis=1, keepdims=True)
    idx = jnp.where(d == rmin, _iota_row(k), jnp.float32(1e9))
    lab = jnp.min(idx, axis=1, keepdims=True)
    return (lab == _iota_row(k)).astype(jnp.float32), lab


def _ident(n):
    return (_iota_col(n) == _iota_row(n)).astype(jnp.float32)


def _split3(v):
    h = v.astype(jnp.bfloat16).astype(jnp.float32)
    m = (v - h).astype(jnp.bfloat16).astype(jnp.float32)
    return h, m, v - h - m


def _gather3(onehot, parts):
    h, m, l = parts
    return (_dot(onehot, h, 0, 0, _DEF) + _dot(onehot, m, 0, 0, _DEF)
            + _dot(onehot, l, 0, 0, _DEF))


def _bf16(v):
    return v.astype(jnp.bfloat16).astype(jnp.float32)


def _body(tok_ref, pinit_ref, scal_ref, w_ref, b_ref, scale_ref, out_ref):
    for j in range(tok_ref.shape[0]):
        anchors_out, cents_out = _pipeline(
            tok_ref, pinit_ref, scal_ref, w_ref, b_ref, scale_ref, j)
        out_ref[j, 0:_KA, :] = anchors_out
        out_ref[j, _KA:_KA + _KC, :] = cents_out


def _pipeline(tok_ref, pinit_ref, scal_ref, w_ref, b_ref, scale_ref, j):
    tok = tok_ref[j]
    scal = scal_ref[...]
    x_col = scal[:, 0:1] + scal[:, 1:2] + scal[:, 2:3]
    y_col = scal[:, 3:4] + scal[:, 4:5] + scal[:, 5:6]
    tok_parts = _split3(tok)

    norms = jnp.sqrt(jnp.sum(tok * tok, axis=1, keepdims=True))
    sal_col = norms / jnp.maximum(jnp.sum(norms), jnp.float32(1e-8))
    sal_row = _dot(sal_col, _ident(_N), 0, 0)
    ii = jax.lax.broadcasted_iota(jnp.int32, (_N, _N), 0)
    jj = jax.lax.broadcasted_iota(jnp.int32, (_N, _N), 1)
    beats = (sal_row > sal_col) | ((sal_row == sal_col) & (jj < ii))
    rank_col = jnp.sum(beats.astype(jnp.float32), axis=1, keepdims=True)

    onehot_a = (rank_col == _iota_row(_KA)).astype(jnp.float32)
    anchors = _gather3(onehot_a, tok_parts)
    ax_row = _dot(x_col, onehot_a, 0, 0)
    ay_row = _dot(y_col, onehot_a, 0, 0)

    notsel_col = (rank_col >= jnp.float32(_KA)).astype(jnp.float32)
    notsel_row = _dot(notsel_col, _ident(_N), 0, 0, _DEF)
    tri = (jj < ii).astype(jnp.float32)
    ctx_rank_col = jnp.sum(tri * notsel_row, axis=1, keepdims=True)
    onehot_c = (ctx_rank_col == _iota_row(_NCTX)).astype(jnp.float32) \
        * notsel_col
    cth = _dot(onehot_c, tok_parts[0], 0, 0, _DEF)
    ctm = _dot(onehot_c, tok_parts[1], 0, 0, _DEF)
    ctl = _dot(onehot_c, tok_parts[2], 0, 0, _DEF)
    ctx_tok = cth + ctm + ctl
    ctx_scal = _dot(onehot_c, scal, 0, 0, _DEF)
    cx_col = ctx_scal[:, 0:1] + ctx_scal[:, 1:2] + ctx_scal[:, 2:3]
    cy_col = ctx_scal[:, 3:4] + ctx_scal[:, 4:5] + ctx_scal[:, 5:6]

    pinit = pinit_ref[j]
    centroids = (_dot(pinit, cth, 1, 0, _DEF) + _dot(pinit, ctm, 1, 0, _DEF)
                 + _dot(pinit, ctl, 1, 0, _DEF))
    pin_scal = _dot(pinit, ctx_scal, 1, 0, _DEF)
    ccx_col = pin_scal[:, 0:1] + pin_scal[:, 1:2] + pin_scal[:, 2:3]
    ccy_col = pin_scal[:, 3:4] + pin_scal[:, 4:5] + pin_scal[:, 5:6]

    tn2_col = jnp.sum(ctx_tok * ctx_tok, axis=1, keepdims=True)
    sn2_col = cx_col * cx_col + cy_col * cy_col
    cxb_col = _bf16(cx_col)
    cyb_col = _bf16(cy_col)
    rhs9 = jnp.concatenate([jnp.ones((_NCTX, 1), jnp.float32), ctx_scal],
                           axis=1)
    ones_row_d = jnp.ones((1, _D), jnp.float32)
    ident_kc = _ident(_KC)

    assign = None
    segres = None
    for _ in range(_NITER):
        cn2_row = _dot(ones_row_d, centroids * centroids, 1, 1)
        dotfc = _dot(ctx_tok, centroids, 1, 1, _DEF)
        fd = jnp.sqrt(jnp.maximum(tn2_col + cn2_row - 2.0 * dotfc, 0.0))
        ccx_row = _dot(ccx_col, ident_kc, 0, 0)
        ccy_row = _dot(ccy_col, ident_kc, 0, 0)
        ccxb_row = _bf16(ccx_row)
        ccyb_row = _bf16(ccy_row)
        cs2_row = ccx_row * ccx_row + ccy_row * ccy_row
        sdot = cxb_col * ccxb_row + cyb_col * ccyb_row
        sd = jnp.sqrt(jnp.maximum(sn2_col + cs2_row - 2.0 * sdot, 0.0))
        assign, _ = _first_argmin_onehot(fd + _SW * sd, _KC)
        segres = _dot(assign, rhs9, 0, 0, _DEF)
        cnt_col = segres[:, 0:1]
        csx_col = segres[:, 1:2] + segres[:, 2:3] + segres[:, 3:4]
        csy_col = segres[:, 4:5] + segres[:, 5:6] + segres[:, 6:7]
        tsum = (_dot(assign, cth, 0, 0, _DEF) + _dot(assign, ctm, 0, 0, _DEF)
                + _dot(assign, ctl, 0, 0, _DEF))
        upd_col = cnt_col > 0.0
        denom_col = jnp.maximum(cnt_col, 1.0)
        centroids = jnp.where(upd_col, tsum / denom_col, centroids)
        ccx_col = jnp.where(upd_col, csx_col / denom_col, ccx_col)
        ccy_col = jnp.where(upd_col, csy_col / denom_col, ccy_col)

    gath = _dot(assign, centroids, 1, 0, _DEF)
    agg = _dot(assign, ctx_tok - gath, 0, 0, _DEF)
    agg = _dot(agg, w_ref[...], 1, 1, _DEF) + b_ref[0:1, :]

    cnt_col = segres[:, 0:1]
    denom_col = jnp.maximum(cnt_col, 1.0)
    sidx_col = 256.0 * segres[:, 7:8] + segres[:, 8:9]
    aidx_col = sidx_col / denom_col
    ccy2_col = jnp.floor(aidx_col / jnp.float32(_SIDE)) / jnp.float32(_SIDE)
    ccx2_col = jnp.mod(aidx_col, jnp.float32(_SIDE)) / jnp.float32(_SIDE)

    c2_col = ccx2_col * ccx2_col + ccy2_col * ccy2_col
    a2_row = ax_row * ax_row + ay_row * ay_row
    ddot = ccx2_col * ax_row + ccy2_col * ay_row
    dd = jnp.sqrt(jnp.maximum(c2_col + a2_row - 2.0 * ddot, 0.0))
    scat, _ = _first_argmin_onehot(dd, _KA)

    contrib = (scale_ref[0, 0] * agg) * (cnt_col > 0.0).astype(jnp.float32)
    inj = _dot(scat, contrib, 0, 0, _DEF)

    return anchors + inj, centroids


def _init_onehots():
    mats = []
    for bi in range(_B):
        perm = jax.random.permutation(
            jax.random.fold_in(jax.random.key(42), bi), _NCTX)[:_KC]
        mats.append(perm[:, None] == jnp.arange(_NCTX)[None, :])
    return jnp.stack(mats).astype(jnp.float32)


def _scalar_sources():
    y = jnp.arange(_SIDE, dtype=jnp.float32) / _SIDE
    x = jnp.arange(_SIDE, dtype=jnp.float32) / _SIDE
    gy, gx = jnp.meshgrid(y, x, indexing='ij')
    coords = jnp.stack([gx, gy], axis=-1).reshape(-1, 2)

    def split3(v):
        h = v.astype(jnp.bfloat16).astype(jnp.float32)
        m = (v - h).astype(jnp.bfloat16).astype(jnp.float32)
        return h, m, v - h - m

    xh, xm, xl = split3(coords[:, 0])
    yh, ym, yl = split3(coords[:, 1])
    idx = jnp.arange(_N, dtype=jnp.float32)
    ihi = jnp.floor(idx / 256.0)
    ilo = idx - 256.0 * ihi
    return jnp.stack([xh, xm, xl, yh, ym, yl, ihi, ilo], axis=-1)


_IMGS_PER_STEP = 2


def _run(visual_tokens, pinit, scal, W, b2, scale2):
    nb = visual_tokens.shape[0]
    g = _IMGS_PER_STEP
    return pl.pallas_call(
        _body,
        grid=(nb // g,),
        in_specs=[
            pl.BlockSpec((g, _N, _D), lambda i: (i, 0, 0)),
            pl.BlockSpec((g, _KC, _NCTX), lambda i: (i, 0, 0)),
            pl.BlockSpec((_N, 8), lambda i: (0, 0)),
            pl.BlockSpec((_D, _D), lambda i: (0, 0)),
            pl.BlockSpec((1, _D), lambda i: (0, 0)),
            pl.BlockSpec((1, 1), lambda i: (0, 0)),
        ],
        out_specs=pl.BlockSpec((g, _KA + _KC, _D), lambda i: (i, 0, 0)),
        out_shape=jax.ShapeDtypeStruct((nb, _KA + _KC, _D), jnp.float32),
        compiler_params=pltpu.CompilerParams(
            dimension_semantics=("parallel",)),
    )(visual_tokens, pinit, scal, W, b2, scale2)


def kernel(visual_tokens, W, b, residual_scale):
    pinit = _init_onehots()
    scal = _scalar_sources()
    compressed = _run(visual_tokens, pinit, scal, W,
                      b.reshape(1, _D), residual_scale.reshape(1, 1))
    attention_mask = jnp.ones((_B, _KA + _KC), dtype=jnp.float32)
    return compressed, attention_mask


# speedup vs baseline: 1.2612x; 1.2612x over previous
import math

import jax
import jax.numpy as jnp
import numpy as np
from jax.experimental import pallas as pl
from jax.experimental.pallas import tpu as pltpu

_D = 768
_B = 4
_N = 576
_SIDE = 24
_KA = 144
_KC = 144
_NCTX = _N - _KA
_NITER = 10
_SW = 0.1

_HI = jax.lax.Precision.HIGHEST
_DEF = jax.lax.Precision.DEFAULT


def _dot(a, b, ca, cb, prec=_HI):
    return jax.lax.dot_general(
        a, b, (((ca,), (cb,)), ((), ())),
        precision=prec, preferred_element_type=jnp.float32)


def _iota_row(n, dtype=jnp.float32):
    return jax.lax.broadcasted_iota(jnp.int32, (1, n), 1).astype(dtype)


def _iota_col(n, dtype=jnp.float32):
    return jax.lax.broadcasted_iota(jnp.int32, (n, 1), 0).astype(dtype)


def _first_argmin_onehot(d, k):
    rmin = jnp.min(d, axis=1, keepdims=True)
    idx = jnp.where(d == rmin, _iota_row(k), jnp.float32(1e9))
    lab = jnp.min(idx, axis=1, keepdims=True)
    return (lab == _iota_row(k)).astype(jnp.float32), lab


def _ident(n):
    return (_iota_col(n) == _iota_row(n)).astype(jnp.float32)


def _split3(v):
    h = v.astype(jnp.bfloat16).astype(jnp.float32)
    m = (v - h).astype(jnp.bfloat16).astype(jnp.float32)
    return h, m, v - h - m


def _gather3(onehot, parts):
    h, m, l = parts
    return (_dot(onehot, h, 0, 0, _DEF) + _dot(onehot, m, 0, 0, _DEF)
            + _dot(onehot, l, 0, 0, _DEF))


def _bf16(v):
    return v.astype(jnp.bfloat16).astype(jnp.float32)


def _body(tok_ref, pinit_ref, scal_ref, w_ref, b_ref, scale_ref, out_ref):
    for j in range(tok_ref.shape[0]):
        anchors_out, cents_out = _pipeline(
            tok_ref, pinit_ref, scal_ref, w_ref, b_ref, scale_ref, j)
        out_ref[j, 0:_KA, :] = anchors_out
        out_ref[j, _KA:_KA + _KC, :] = cents_out


def _pipeline(tok_ref, pinit_ref, scal_ref, w_ref, b_ref, scale_ref, j):
    tok = tok_ref[j]
    scal = scal_ref[...]
    x_col = scal[:, 0:1] + scal[:, 1:2] + scal[:, 2:3]
    y_col = scal[:, 3:4] + scal[:, 4:5] + scal[:, 5:6]
    tok_parts = _split3(tok)

    norms = jnp.sqrt(jnp.sum(tok * tok, axis=1, keepdims=True))
    sal_col = norms / jnp.maximum(jnp.sum(norms), jnp.float32(1e-8))
    sal_row = _dot(sal_col, _ident(_N), 0, 0)
    ii = jax.lax.broadcasted_iota(jnp.int32, (_N, _N), 0)
    jj = jax.lax.broadcasted_iota(jnp.int32, (_N, _N), 1)
    beats = (sal_row > sal_col) | ((sal_row == sal_col) & (jj < ii))
    rank_col = jnp.sum(beats.astype(jnp.float32), axis=1, keepdims=True)

    onehot_a = (rank_col == _iota_row(_KA)).astype(jnp.float32)
    anchors = _gather3(onehot_a, tok_parts)
    ax_row = _dot(x_col, onehot_a, 0, 0)
    ay_row = _dot(y_col, onehot_a, 0, 0)

    notsel_col = (rank_col >= jnp.float32(_KA)).astype(jnp.float32)
    notsel_row = _dot(notsel_col, _ident(_N), 0, 0, _DEF)
    tri = (jj < ii).astype(jnp.float32)
    ctx_rank_col = jnp.sum(tri * notsel_row, axis=1, keepdims=True)
    onehot_c = (ctx_rank_col == _iota_row(_NCTX)).astype(jnp.float32) \
        * notsel_col
    cth = _dot(onehot_c, tok_parts[0], 0, 0, _DEF)
    ctm = _dot(onehot_c, tok_parts[1], 0, 0, _DEF)
    ctl = _dot(onehot_c, tok_parts[2], 0, 0, _DEF)
    ctx_tok = cth + ctm + ctl
    ctx_scal = _dot(onehot_c, scal, 0, 0, _DEF)
    cx_col = ctx_scal[:, 0:1] + ctx_scal[:, 1:2] + ctx_scal[:, 2:3]
    cy_col = ctx_scal[:, 3:4] + ctx_scal[:, 4:5] + ctx_scal[:, 5:6]

    pinit = pinit_ref[j]
    centroids = (_dot(pinit, cth, 1, 0, _DEF) + _dot(pinit, ctm, 1, 0, _DEF)
                 + _dot(pinit, ctl, 1, 0, _DEF))
    pin_scal = _dot(pinit, ctx_scal, 1, 0, _DEF)
    ccx_col = pin_scal[:, 0:1] + pin_scal[:, 1:2] + pin_scal[:, 2:3]
    ccy_col = pin_scal[:, 3:4] + pin_scal[:, 4:5] + pin_scal[:, 5:6]

    tn2_col = jnp.sum(ctx_tok * ctx_tok, axis=1, keepdims=True)
    sn2_col = cx_col * cx_col + cy_col * cy_col
    cxb_col = _bf16(cx_col)
    cyb_col = _bf16(cy_col)
    rhs9 = jnp.concatenate([jnp.ones((_NCTX, 1), jnp.float32), ctx_scal],
                           axis=1)
    ones_row_d = jnp.ones((1, _D), jnp.float32)
    ident_kc = _ident(_KC)

    assign = None
    segres = None
    for _ in range(_NITER):
        cn2_row = _dot(ones_row_d, centroids * centroids, 1, 1)
        dotfc = _dot(ctx_tok, centroids, 1, 1, _DEF)
        fd = jnp.sqrt(jnp.maximum(tn2_col + cn2_row - 2.0 * dotfc, 0.0))
        ccx_row = _dot(ccx_col, ident_kc, 0, 0)
        ccy_row = _dot(ccy_col, ident_kc, 0, 0)
        ccxb_row = _bf16(ccx_row)
        ccyb_row = _bf16(ccy_row)
        cs2_row = ccx_row * ccx_row + ccy_row * ccy_row
        sdot = cxb_col * ccxb_row + cyb_col * ccyb_row
        sd = jnp.sqrt(jnp.maximum(sn2_col + cs2_row - 2.0 * sdot, 0.0))
        assign, _ = _first_argmin_onehot(fd + _SW * sd, _KC)
        segres = _dot(assign, rhs9, 0, 0, _DEF)
        cnt_col = segres[:, 0:1]
        csx_col = segres[:, 1:2] + segres[:, 2:3] + segres[:, 3:4]
        csy_col = segres[:, 4:5] + segres[:, 5:6] + segres[:, 6:7]
        tsum = (_dot(assign, cth, 0, 0, _DEF) + _dot(assign, ctm, 0, 0, _DEF)
                + _dot(assign, ctl, 0, 0, _DEF))
        upd_col = cnt_col > 0.0
        denom_col = jnp.maximum(cnt_col, 1.0)
        centroids = jnp.where(upd_col, tsum / denom_col, centroids)
        ccx_col = jnp.where(upd_col, csx_col / denom_col, ccx_col)
        ccy_col = jnp.where(upd_col, csy_col / denom_col, ccy_col)

    gath = _dot(assign, centroids, 1, 0, _DEF)
    agg = _dot(assign, ctx_tok - gath, 0, 0, _DEF)
    agg = _dot(agg, w_ref[...], 1, 1, _DEF) + b_ref[0:1, :]

    cnt_col = segres[:, 0:1]
    denom_col = jnp.maximum(cnt_col, 1.0)
    sidx_col = 256.0 * segres[:, 7:8] + segres[:, 8:9]
    aidx_col = sidx_col / denom_col
    ccy2_col = jnp.floor(aidx_col / jnp.float32(_SIDE)) / jnp.float32(_SIDE)
    ccx2_col = jnp.mod(aidx_col, jnp.float32(_SIDE)) / jnp.float32(_SIDE)

    c2_col = ccx2_col * ccx2_col + ccy2_col * ccy2_col
    a2_row = ax_row * ax_row + ay_row * ay_row
    ddot = ccx2_col * ax_row + ccy2_col * ay_row
    dd = jnp.sqrt(jnp.maximum(c2_col + a2_row - 2.0 * ddot, 0.0))
    scat, _ = _first_argmin_onehot(dd, _KA)

    contrib = (scale_ref[0, 0] * agg) * (cnt_col > 0.0).astype(jnp.float32)
    inj = _dot(scat, contrib, 0, 0, _DEF)

    return anchors + inj, centroids


def _init_onehots():
    mats = []
    for bi in range(_B):
        perm = jax.random.permutation(
            jax.random.fold_in(jax.random.key(42), bi), _NCTX)[:_KC]
        mats.append(perm[:, None] == jnp.arange(_NCTX)[None, :])
    return jnp.stack(mats).astype(jnp.float32)


def _scalar_sources():
    y = jnp.arange(_SIDE, dtype=jnp.float32) / _SIDE
    x = jnp.arange(_SIDE, dtype=jnp.float32) / _SIDE
    gy, gx = jnp.meshgrid(y, x, indexing='ij')
    coords = jnp.stack([gx, gy], axis=-1).reshape(-1, 2)

    def split3(v):
        h = v.astype(jnp.bfloat16).astype(jnp.float32)
        m = (v - h).astype(jnp.bfloat16).astype(jnp.float32)
        return h, m, v - h - m

    xh, xm, xl = split3(coords[:, 0])
    yh, ym, yl = split3(coords[:, 1])
    idx = jnp.arange(_N, dtype=jnp.float32)
    ihi = jnp.floor(idx / 256.0)
    ilo = idx - 256.0 * ihi
    return jnp.stack([xh, xm, xl, yh, ym, yl, ihi, ilo], axis=-1)


_IMGS_PER_STEP = 1


def _run(visual_tokens, pinit, scal, W, b2, scale2):
    nb = visual_tokens.shape[0]
    g = _IMGS_PER_STEP
    return pl.pallas_call(
        _body,
        grid=(nb // g,),
        in_specs=[
            pl.BlockSpec((g, _N, _D), lambda i: (i, 0, 0)),
            pl.BlockSpec((g, _KC, _NCTX), lambda i: (i, 0, 0)),
            pl.BlockSpec((_N, 8), lambda i: (0, 0)),
            pl.BlockSpec((_D, _D), lambda i: (0, 0)),
            pl.BlockSpec((1, _D), lambda i: (0, 0)),
            pl.BlockSpec((1, 1), lambda i: (0, 0)),
        ],
        out_specs=pl.BlockSpec((g, _KA + _KC, _D), lambda i: (i, 0, 0)),
        out_shape=jax.ShapeDtypeStruct((nb, _KA + _KC, _D), jnp.float32),
        compiler_params=pltpu.CompilerParams(
            dimension_semantics=("parallel",)),
    )(visual_tokens, pinit, scal, W, b2, scale2)


def kernel(visual_tokens, W, b, residual_scale):
    pinit = _init_onehots()
    scal = _scalar_sources()
    compressed = _run(visual_tokens, pinit, scal, W,
                      b.reshape(1, _D), residual_scale.reshape(1, 1))
    attention_mask = jnp.ones((_B, _KA + _KC), dtype=jnp.float32)
    return compressed, attention_mask


# speedup vs baseline: 1.3978x; 1.1083x over previous
import math

import jax
import jax.numpy as jnp
import numpy as np
from jax.experimental import pallas as pl
from jax.experimental.pallas import tpu as pltpu

_D = 768
_B = 4
_N = 576
_SIDE = 24
_KA = 144
_KC = 144
_NCTX = _N - _KA
_NITER = 10
_SW = 0.1

_HI = jax.lax.Precision.HIGHEST
_DEF = jax.lax.Precision.DEFAULT


def _dot(a, b, ca, cb, prec=_HI):
    return jax.lax.dot_general(
        a, b, (((ca,), (cb,)), ((), ())),
        precision=prec, preferred_element_type=jnp.float32)


def _iota_row(n, dtype=jnp.float32):
    return jax.lax.broadcasted_iota(jnp.int32, (1, n), 1).astype(dtype)


def _iota_col(n, dtype=jnp.float32):
    return jax.lax.broadcasted_iota(jnp.int32, (n, 1), 0).astype(dtype)


def _first_argmin_onehot(d, k):
    rmin = jnp.min(d, axis=1, keepdims=True)
    idx = jnp.where(d == rmin, _iota_row(k), jnp.float32(1e9))
    lab = jnp.min(idx, axis=1, keepdims=True)
    return (lab == _iota_row(k)).astype(jnp.float32), lab


def _ident(n):
    return (_iota_col(n) == _iota_row(n)).astype(jnp.float32)


def _split3(v):
    h = v.astype(jnp.bfloat16).astype(jnp.float32)
    m = (v - h).astype(jnp.bfloat16).astype(jnp.float32)
    return h, m, v - h - m


def _gather3(onehot, parts):
    h, m, l = parts
    return (_dot(onehot, h, 0, 0, _DEF) + _dot(onehot, m, 0, 0, _DEF)
            + _dot(onehot, l, 0, 0, _DEF))


def _bf16(v):
    return v.astype(jnp.bfloat16).astype(jnp.float32)


def _body(tok_ref, pinit_ref, scal_ref, w_ref, b_ref, scale_ref, out_ref):
    for j in range(tok_ref.shape[0]):
        anchors_out, cents_out = _pipeline(
            tok_ref, pinit_ref, scal_ref, w_ref, b_ref, scale_ref, j)
        out_ref[j, 0:_KA, :] = anchors_out
        out_ref[j, _KA:_KA + _KC, :] = cents_out


def _pipeline(tok_ref, pinit_ref, scal_ref, w_ref, b_ref, scale_ref, j):
    tok = tok_ref[j]
    scal = scal_ref[...]
    x_col = scal[:, 0:1] + scal[:, 1:2] + scal[:, 2:3]
    y_col = scal[:, 3:4] + scal[:, 4:5] + scal[:, 5:6]
    tok_parts = _split3(tok)

    norms = jnp.sqrt(jnp.sum(tok * tok, axis=1, keepdims=True))
    sal_col = norms / jnp.maximum(jnp.sum(norms), jnp.float32(1e-8))
    sal_row = _dot(sal_col, _ident(_N), 0, 0)
    ii = jax.lax.broadcasted_iota(jnp.int32, (_N, _N), 0)
    jj = jax.lax.broadcasted_iota(jnp.int32, (_N, _N), 1)
    beats = (sal_row > sal_col) | ((sal_row == sal_col) & (jj < ii))
    rank_col = jnp.sum(beats.astype(jnp.float32), axis=1, keepdims=True)

    onehot_a = (rank_col == _iota_row(_KA)).astype(jnp.float32)
    anchors = _gather3(onehot_a, tok_parts)
    ax_row = _dot(x_col, onehot_a, 0, 0)
    ay_row = _dot(y_col, onehot_a, 0, 0)

    notsel_col = (rank_col >= jnp.float32(_KA)).astype(jnp.float32)
    notsel_row = _dot(notsel_col, _ident(_N), 0, 0, _DEF)
    tri = (jj < ii).astype(jnp.float32)
    ctx_rank_col = jnp.sum(tri * notsel_row, axis=1, keepdims=True)
    onehot_c = (ctx_rank_col == _iota_row(_NCTX)).astype(jnp.float32) \
        * notsel_col
    cth = _dot(onehot_c, tok_parts[0], 0, 0, _DEF)
    ctm = _dot(onehot_c, tok_parts[1], 0, 0, _DEF)
    ctl = _dot(onehot_c, tok_parts[2], 0, 0, _DEF)
    ctx_tok = cth + ctm + ctl
    ctx_scal = _dot(onehot_c, scal, 0, 0, _DEF)
    cx_col = ctx_scal[:, 0:1] + ctx_scal[:, 1:2] + ctx_scal[:, 2:3]
    cy_col = ctx_scal[:, 3:4] + ctx_scal[:, 4:5] + ctx_scal[:, 5:6]

    pinit = pinit_ref[j]
    centroids = (_dot(pinit, cth, 1, 0, _DEF) + _dot(pinit, ctm, 1, 0, _DEF)
                 + _dot(pinit, ctl, 1, 0, _DEF))
    pin_scal = _dot(pinit, ctx_scal, 1, 0, _DEF)
    ccx_col = pin_scal[:, 0:1] + pin_scal[:, 1:2] + pin_scal[:, 2:3]
    ccy_col = pin_scal[:, 3:4] + pin_scal[:, 4:5] + pin_scal[:, 5:6]

    tn2_col = jnp.sum(ctx_tok * ctx_tok, axis=1, keepdims=True)
    sn2_col = cx_col * cx_col + cy_col * cy_col
    cxb_col = _bf16(cx_col)
    cyb_col = _bf16(cy_col)
    rhs9 = jnp.concatenate([jnp.ones((_NCTX, 1), jnp.float32), ctx_scal],
                           axis=1)

    assign = None
    segres = None
    for _ in range(_NITER):
        cn2_col = jnp.sum(centroids * centroids, axis=1, keepdims=True)
        cn2_row = jnp.transpose(cn2_col)
        dotfc = _dot(ctx_tok, centroids, 1, 1, _DEF)
        fd = jnp.sqrt(jnp.maximum(tn2_col + cn2_row - 2.0 * dotfc, 0.0))
        ccxb_row = jnp.transpose(_bf16(ccx_col))
        ccyb_row = jnp.transpose(_bf16(ccy_col))
        cs2_row = jnp.transpose(ccx_col * ccx_col + ccy_col * ccy_col)
        sdot = cxb_col * ccxb_row + cyb_col * ccyb_row
        sd = jnp.sqrt(jnp.maximum(sn2_col + cs2_row - 2.0 * sdot, 0.0))
        assign, _ = _first_argmin_onehot(fd + _SW * sd, _KC)
        segres = _dot(assign, rhs9, 0, 0, _DEF)
        cnt_col = segres[:, 0:1]
        csx_col = segres[:, 1:2] + segres[:, 2:3] + segres[:, 3:4]
        csy_col = segres[:, 4:5] + segres[:, 5:6] + segres[:, 6:7]
        tsum = (_dot(assign, cth, 0, 0, _DEF) + _dot(assign, ctm, 0, 0, _DEF)
                + _dot(assign, ctl, 0, 0, _DEF))
        upd_col = cnt_col > 0.0
        denom_col = jnp.maximum(cnt_col, 1.0)
        centroids = jnp.where(upd_col, tsum / denom_col, centroids)
        ccx_col = jnp.where(upd_col, csx_col / denom_col, ccx_col)
        ccy_col = jnp.where(upd_col, csy_col / denom_col, ccy_col)

    gath = _dot(assign, centroids, 1, 0, _DEF)
    agg = _dot(assign, ctx_tok - gath, 0, 0, _DEF)
    agg = _dot(agg, w_ref[...], 1, 1, _DEF) + b_ref[0:1, :]

    cnt_col = segres[:, 0:1]
    denom_col = jnp.maximum(cnt_col, 1.0)
    sidx_col = 256.0 * segres[:, 7:8] + segres[:, 8:9]
    aidx_col = sidx_col / denom_col
    ccy2_col = jnp.floor(aidx_col / jnp.float32(_SIDE)) / jnp.float32(_SIDE)
    ccx2_col = jnp.mod(aidx_col, jnp.float32(_SIDE)) / jnp.float32(_SIDE)

    c2_col = ccx2_col * ccx2_col + ccy2_col * ccy2_col
    a2_row = ax_row * ax_row + ay_row * ay_row
    ddot = ccx2_col * ax_row + ccy2_col * ay_row
    dd = jnp.sqrt(jnp.maximum(c2_col + a2_row - 2.0 * ddot, 0.0))
    scat, _ = _first_argmin_onehot(dd, _KA)

    contrib = (scale_ref[0, 0] * agg) * (cnt_col > 0.0).astype(jnp.float32)
    inj = _dot(scat, contrib, 0, 0, _DEF)

    return anchors + inj, centroids


def _init_onehots():
    mats = []
    for bi in range(_B):
        perm = jax.random.permutation(
            jax.random.fold_in(jax.random.key(42), bi), _NCTX)[:_KC]
        mats.append(perm[:, None] == jnp.arange(_NCTX)[None, :])
    return jnp.stack(mats).astype(jnp.float32)


def _scalar_sources():
    y = jnp.arange(_SIDE, dtype=jnp.float32) / _SIDE
    x = jnp.arange(_SIDE, dtype=jnp.float32) / _SIDE
    gy, gx = jnp.meshgrid(y, x, indexing='ij')
    coords = jnp.stack([gx, gy], axis=-1).reshape(-1, 2)

    def split3(v):
        h = v.astype(jnp.bfloat16).astype(jnp.float32)
        m = (v - h).astype(jnp.bfloat16).astype(jnp.float32)
        return h, m, v - h - m

    xh, xm, xl = split3(coords[:, 0])
    yh, ym, yl = split3(coords[:, 1])
    idx = jnp.arange(_N, dtype=jnp.float32)
    ihi = jnp.floor(idx / 256.0)
    ilo = idx - 256.0 * ihi
    return jnp.stack([xh, xm, xl, yh, ym, yl, ihi, ilo], axis=-1)


_IMGS_PER_STEP = 1


def _run(visual_tokens, pinit, scal, W, b2, scale2):
    nb = visual_tokens.shape[0]
    g = _IMGS_PER_STEP
    return pl.pallas_call(
        _body,
        grid=(nb // g,),
        in_specs=[
            pl.BlockSpec((g, _N, _D), lambda i: (i, 0, 0)),
            pl.BlockSpec((g, _KC, _NCTX), lambda i: (i, 0, 0)),
            pl.BlockSpec((_N, 8), lambda i: (0, 0)),
            pl.BlockSpec((_D, _D), lambda i: (0, 0)),
            pl.BlockSpec((1, _D), lambda i: (0, 0)),
            pl.BlockSpec((1, 1), lambda i: (0, 0)),
        ],
        out_specs=pl.BlockSpec((g, _KA + _KC, _D), lambda i: (i, 0, 0)),
        out_shape=jax.ShapeDtypeStruct((nb, _KA + _KC, _D), jnp.float32),
        compiler_params=pltpu.CompilerParams(
            dimension_semantics=("parallel",)),
    )(visual_tokens, pinit, scal, W, b2, scale2)


def kernel(visual_tokens, W, b, residual_scale):
    pinit = _init_onehots()
    scal = _scalar_sources()
    compressed = _run(visual_tokens, pinit, scal, W,
                      b.reshape(1, _D), residual_scale.reshape(1, 1))
    attention_mask = jnp.ones((_B, _KA + _KC), dtype=jnp.float32)
    return compressed, attention_mask


# speedup vs baseline: 1.4769x; 1.0566x over previous
import math

import jax
import jax.numpy as jnp
import numpy as np
from jax.experimental import pallas as pl
from jax.experimental.pallas import tpu as pltpu

_D = 768
_B = 4
_N = 576
_SIDE = 24
_KA = 144
_KC = 144
_NCTX = _N - _KA
_NITER = 10
_SW = 0.1

_HI = jax.lax.Precision.HIGHEST
_DEF = jax.lax.Precision.DEFAULT


def _dot(a, b, ca, cb, prec=_HI):
    return jax.lax.dot_general(
        a, b, (((ca,), (cb,)), ((), ())),
        precision=prec, preferred_element_type=jnp.float32)


def _iota_row(n, dtype=jnp.float32):
    return jax.lax.broadcasted_iota(jnp.int32, (1, n), 1).astype(dtype)


def _iota_col(n, dtype=jnp.float32):
    return jax.lax.broadcasted_iota(jnp.int32, (n, 1), 0).astype(dtype)


def _first_argmin_onehot(d, k):
    rmin = jnp.min(d, axis=1, keepdims=True)
    idx = jnp.where(d == rmin, _iota_row(k), jnp.float32(1e9))
    lab = jnp.min(idx, axis=1, keepdims=True)
    return (lab == _iota_row(k)).astype(jnp.float32), lab


def _ident(n):
    return (_iota_col(n) == _iota_row(n)).astype(jnp.float32)


def _split3(v):
    h = v.astype(jnp.bfloat16).astype(jnp.float32)
    m = (v - h).astype(jnp.bfloat16).astype(jnp.float32)
    return h, m, v - h - m


def _gather3(onehot, parts):
    h, m, l = parts
    return (_dot(onehot, h, 0, 0, _DEF) + _dot(onehot, m, 0, 0, _DEF)
            + _dot(onehot, l, 0, 0, _DEF))


def _bf16(v):
    return v.astype(jnp.bfloat16).astype(jnp.float32)


def _body(tok_ref, pinit_ref, scal_ref, w_ref, b_ref, scale_ref, out_ref):
    for j in range(tok_ref.shape[0]):
        anchors_out, cents_out = _pipeline(
            tok_ref, pinit_ref, scal_ref, w_ref, b_ref, scale_ref, j)
        out_ref[j, 0:_KA, :] = anchors_out
        out_ref[j, _KA:_KA + _KC, :] = cents_out


def _pipeline(tok_ref, pinit_ref, scal_ref, w_ref, b_ref, scale_ref, j):
    tok = tok_ref[j]
    scal = scal_ref[...]
    x_col = scal[:, 0:1] + scal[:, 1:2] + scal[:, 2:3]
    y_col = scal[:, 3:4] + scal[:, 4:5] + scal[:, 5:6]
    tok_parts = _split3(tok)

    norms = jnp.sqrt(jnp.sum(tok * tok, axis=1, keepdims=True))
    sal_col = norms / jnp.maximum(jnp.sum(norms), jnp.float32(1e-8))
    sal_row = jnp.transpose(sal_col)
    ii = jax.lax.broadcasted_iota(jnp.int32, (_N, _N), 0)
    jj = jax.lax.broadcasted_iota(jnp.int32, (_N, _N), 1)
    beats = (sal_row > sal_col) | ((sal_row == sal_col) & (jj < ii))
    rank_col = jnp.sum(beats.astype(jnp.float32), axis=1, keepdims=True)

    onehot_a = (rank_col == _iota_row(_KA)).astype(jnp.float32)
    anchors = _gather3(onehot_a, tok_parts)
    a_scal = _dot(onehot_a, scal, 0, 0, _DEF)
    ax_row = jnp.transpose(a_scal[:, 0:1] + a_scal[:, 1:2] + a_scal[:, 2:3])
    ay_row = jnp.transpose(a_scal[:, 3:4] + a_scal[:, 4:5] + a_scal[:, 5:6])

    notsel_col = (rank_col >= jnp.float32(_KA)).astype(jnp.float32)
    notsel_row = jnp.transpose(notsel_col)
    tri = (jj < ii).astype(jnp.float32)
    ctx_rank_col = jnp.sum(tri * notsel_row, axis=1, keepdims=True)
    onehot_c = (ctx_rank_col == _iota_row(_NCTX)).astype(jnp.float32) \
        * notsel_col
    cth = _dot(onehot_c, tok_parts[0], 0, 0, _DEF)
    ctm = _dot(onehot_c, tok_parts[1], 0, 0, _DEF)
    ctl = _dot(onehot_c, tok_parts[2], 0, 0, _DEF)
    ctx_tok = cth + ctm + ctl
    ctx_scal = _dot(onehot_c, scal, 0, 0, _DEF)
    cx_col = ctx_scal[:, 0:1] + ctx_scal[:, 1:2] + ctx_scal[:, 2:3]
    cy_col = ctx_scal[:, 3:4] + ctx_scal[:, 4:5] + ctx_scal[:, 5:6]

    pinit = pinit_ref[j]
    centroids = (_dot(pinit, cth, 1, 0, _DEF) + _dot(pinit, ctm, 1, 0, _DEF)
                 + _dot(pinit, ctl, 1, 0, _DEF))
    pin_scal = _dot(pinit, ctx_scal, 1, 0, _DEF)
    ccx_col = pin_scal[:, 0:1] + pin_scal[:, 1:2] + pin_scal[:, 2:3]
    ccy_col = pin_scal[:, 3:4] + pin_scal[:, 4:5] + pin_scal[:, 5:6]

    tn2_col = jnp.sum(ctx_tok * ctx_tok, axis=1, keepdims=True)
    sn2_col = cx_col * cx_col + cy_col * cy_col
    cxb_col = _bf16(cx_col)
    cyb_col = _bf16(cy_col)
    rhs9 = jnp.concatenate([jnp.ones((_NCTX, 1), jnp.float32), ctx_scal],
                           axis=1)

    assign = None
    segres = None
    for _ in range(_NITER):
        cn2_col = jnp.sum(centroids * centroids, axis=1, keepdims=True)
        cn2_row = jnp.transpose(cn2_col)
        dotfc = _dot(ctx_tok, centroids, 1, 1, _DEF)
        fd = jnp.sqrt(jnp.maximum(tn2_col + cn2_row - 2.0 * dotfc, 0.0))
        ccxb_row = jnp.transpose(_bf16(ccx_col))
        ccyb_row = jnp.transpose(_bf16(ccy_col))
        cs2_row = jnp.transpose(ccx_col * ccx_col + ccy_col * ccy_col)
        sdot = cxb_col * ccxb_row + cyb_col * ccyb_row
        sd = jnp.sqrt(jnp.maximum(sn2_col + cs2_row - 2.0 * sdot, 0.0))
        assign, _ = _first_argmin_onehot(fd + _SW * sd, _KC)
        segres = _dot(assign, rhs9, 0, 0, _DEF)
        cnt_col = segres[:, 0:1]
        csx_col = segres[:, 1:2] + segres[:, 2:3] + segres[:, 3:4]
        csy_col = segres[:, 4:5] + segres[:, 5:6] + segres[:, 6:7]
        tsum = (_dot(assign, cth, 0, 0, _DEF) + _dot(assign, ctm, 0, 0, _DEF)
                + _dot(assign, ctl, 0, 0, _DEF))
        upd_col = cnt_col > 0.0
        denom_col = jnp.maximum(cnt_col, 1.0)
        centroids = jnp.where(upd_col, tsum / denom_col, centroids)
        ccx_col = jnp.where(upd_col, csx_col / denom_col, ccx_col)
        ccy_col = jnp.where(upd_col, csy_col / denom_col, ccy_col)

    gath = _dot(assign, centroids, 1, 0, _DEF)
    agg = _dot(assign, ctx_tok - gath, 0, 0, _DEF)
    agg = _dot(agg, w_ref[...], 1, 1, _DEF) + b_ref[0:1, :]

    cnt_col = segres[:, 0:1]
    denom_col = jnp.maximum(cnt_col, 1.0)
    sidx_col = 256.0 * segres[:, 7:8] + segres[:, 8:9]
    aidx_col = sidx_col / denom_col
    ccy2_col = jnp.floor(aidx_col / jnp.float32(_SIDE)) / jnp.float32(_SIDE)
    ccx2_col = jnp.mod(aidx_col, jnp.float32(_SIDE)) / jnp.float32(_SIDE)

    c2_col = ccx2_col * ccx2_col + ccy2_col * ccy2_col
    a2_row = ax_row * ax_row + ay_row * ay_row
    ddot = ccx2_col * ax_row + ccy2_col * ay_row
    dd = jnp.sqrt(jnp.maximum(c2_col + a2_row - 2.0 * ddot, 0.0))
    scat, _ = _first_argmin_onehot(dd, _KA)

    contrib = (scale_ref[0, 0] * agg) * (cnt_col > 0.0).astype(jnp.float32)
    inj = _dot(scat, contrib, 0, 0, _DEF)

    return anchors + inj, centroids


def _init_onehots():
    mats = []
    for bi in range(_B):
        perm = jax.random.permutation(
            jax.random.fold_in(jax.random.key(42), bi), _NCTX)[:_KC]
        mats.append(perm[:, None] == jnp.arange(_NCTX)[None, :])
    return jnp.stack(mats).astype(jnp.float32)


def _scalar_sources():
    y = jnp.arange(_SIDE, dtype=jnp.float32) / _SIDE
    x = jnp.arange(_SIDE, dtype=jnp.float32) / _SIDE
    gy, gx = jnp.meshgrid(y, x, indexing='ij')
    coords = jnp.stack([gx, gy], axis=-1).reshape(-1, 2)

    def split3(v):
        h = v.astype(jnp.bfloat16).astype(jnp.float32)
        m = (v - h).astype(jnp.bfloat16).astype(jnp.float32)
        return h, m, v - h - m

    xh, xm, xl = split3(coords[:, 0])
    yh, ym, yl = split3(coords[:, 1])
    idx = jnp.arange(_N, dtype=jnp.float32)
    ihi = jnp.floor(idx / 256.0)
    ilo = idx - 256.0 * ihi
    return jnp.stack([xh, xm, xl, yh, ym, yl, ihi, ilo], axis=-1)


_IMGS_PER_STEP = 1


def _run(visual_tokens, pinit, scal, W, b2, scale2):
    nb = visual_tokens.shape[0]
    g = _IMGS_PER_STEP
    return pl.pallas_call(
        _body,
        grid=(nb // g,),
        in_specs=[
            pl.BlockSpec((g, _N, _D), lambda i: (i, 0, 0)),
            pl.BlockSpec((g, _KC, _NCTX), lambda i: (i, 0, 0)),
            pl.BlockSpec((_N, 8), lambda i: (0, 0)),
            pl.BlockSpec((_D, _D), lambda i: (0, 0)),
            pl.BlockSpec((1, _D), lambda i: (0, 0)),
            pl.BlockSpec((1, 1), lambda i: (0, 0)),
        ],
        out_specs=pl.BlockSpec((g, _KA + _KC, _D), lambda i: (i, 0, 0)),
        out_shape=jax.ShapeDtypeStruct((nb, _KA + _KC, _D), jnp.float32),
        compiler_params=pltpu.CompilerParams(
            dimension_semantics=("parallel",)),
    )(visual_tokens, pinit, scal, W, b2, scale2)


def kernel(visual_tokens, W, b, residual_scale):
    pinit = _init_onehots()
    scal = _scalar_sources()
    compressed = _run(visual_tokens, pinit, scal, W,
                      b.reshape(1, _D), residual_scale.reshape(1, 1))
    attention_mask = jnp.ones((_B, _KA + _KC), dtype=jnp.float32)
    return compressed, attention_mask
